# baseline (speedup 1.0000x reference)
import functools

import jax
import jax.numpy as jnp
from jax import lax
from jax.experimental import pallas as pl
from jax.experimental.pallas import tpu as pltpu
from jax.experimental.pallas import tpu_sc as plsc

_NC = 2
_NS = 16
_NW = _NC * _NS

_SEG = 128
_GROUP_ROWS = 4 * _SEG

_PACK = 4
_BQ = 8192
_NCH = 5
_BT = 8192


def _tc_table_prep(embt, v, d):
    def prep_kernel(i_ref, o_ref):
        o_ref[:, :d] = jnp.transpose(i_ref[...])

    return pl.pallas_call(
        prep_kernel,
        grid=((v + _BT - 1) // _BT,),
        in_specs=[pl.BlockSpec((d, _BT), lambda i: (0, i))],
        out_specs=pl.BlockSpec((_BT, 128), lambda i: (i, 0)),
        out_shape=jax.ShapeDtypeStruct((v, 128), jnp.float32),
    )(embt)


def _sc_gather_chunk(table128, idx, chunk, n_chunks, d):
    n = idx.shape[0]
    q = n // _PACK
    n_groups = q // _SEG
    gpc = n_groups // n_chunks
    gpw = gpc // _NW
    g0 = chunk * gpc
    mesh = plsc.VectorSubcoreMesh(core_axis_name="c", subcore_axis_name="s")

    @functools.partial(
        pl.kernel,
        out_type=jax.ShapeDtypeStruct((n // n_chunks, d), jnp.float32),
        mesh=mesh,
        scratch_types=[
            pltpu.VMEM((_PACK * _SEG,), jnp.int32),
            pltpu.VMEM((_GROUP_ROWS,), jnp.int32),
            pltpu.VMEM((_GROUP_ROWS, 128), jnp.float32),
            pltpu.VMEM((_GROUP_ROWS, d), jnp.float32),
            pltpu.SemaphoreType.DMA,
        ],
        compiler_params=pltpu.CompilerParams(
            use_tc_tiling_on_sc=False, needs_layout_passes=False
        ),
    )
    def gather_kernel(table_hbm, idx_hbm, out_hbm, idx_v, ilv_v, rows_v,
                      cmp_v, sem):
        wid = lax.axis_index("s") * _NC + lax.axis_index("c")
        lane = lax.broadcasted_iota(jnp.int32, (16,), 0)
        half = d // 2

        def body(i, carry):
            gg = g0 + wid * gpw + i
            p0 = gg * _SEG
            blk = p0 // _BQ
            r0 = p0 % _BQ
            tok0 = blk * (_PACK * _BQ) + r0
            for c in range(_PACK):
                pltpu.sync_copy(
                    idx_hbm.at[pl.ds(tok0 + c * _BQ, _SEG)],
                    idx_v.at[pl.ds(c * _SEG, _SEG)],
                )
            for c in range(_PACK):
                for j in range(_SEG // 16):
                    vv = idx_v[pl.ds(c * _SEG + j * 16, 16)]
                    dst = (j * 16 + lane) * _PACK + c
                    plsc.store_scatter(ilv_v, [dst], vv)
            copies = [
                pltpu.async_copy(
                    table_hbm.at[ilv_v.at[pl.ds(b * 128, 128)]],
                    rows_v.at[pl.ds(b * 128, 128)],
                    sem,
                )
                for b in range(_GROUP_ROWS // 128)
            ]
            for cp in copies:
                cp.wait()

            def extract(t, carry2):
                cmp_v[t, pl.ds(0, half)] = rows_v[t, pl.ds(0, half)]
                cmp_v[t, pl.ds(half, half)] = rows_v[t, pl.ds(half, half)]
                return carry2

            lax.fori_loop(0, _GROUP_ROWS, extract, 0)

            lp0 = (gg - g0) * _SEG
            pltpu.sync_copy(
                cmp_v, out_hbm.at[pl.ds(lp0 * _PACK, _GROUP_ROWS)]
            )
            return carry

        lax.fori_loop(0, gpw, body, 0)

    return gather_kernel(table128, idx)


def _tc_mlp_chunk(e4, w1k, b1kt, w2k, b2kt, sk, tags, acc, col0, out_shape):
    qc, dk = e4.shape
    hk = w1k.shape[1]
    hp = hk // _PACK
    grid = qc // _BQ
    blk0 = col0 // (_PACK * _BQ)

    def mlp_kernel(*refs):
        if acc is not None:
            refs = refs[1:]
        e_ref, w1_ref, b1_ref, w2_ref, b2_ref, s_ref, o_ref = refs
        e = e_ref[...]
        hidt = lax.dot_general(
            w1_ref[...], e, (((0,), (1,)), ((), ())),
            preferred_element_type=jnp.float32,
        )
        hidt = jnp.maximum(hidt + b1_ref[...], 0.0)
        t4t = lax.dot_general(
            w2_ref[...], hidt, (((0,), (0,)), ((), ())),
            preferred_element_type=jnp.float32,
        )
        t4t = t4t + b2_ref[...]
        m = jnp.max(t4t)
        ext = jnp.exp(t4t - m)
        sumst = lax.dot_general(
            s_ref[...], ext, (((0,), (0,)), ((), ())),
            preferred_element_type=jnp.float32,
        )
        rt = t4t - (m + jnp.log(sumst))
        for g in range(_PACK):
            o_ref[:, pl.ds(g * _BQ, _BQ)] = rt[g * hp:g * hp + tags, :]

    specs = [
        pl.BlockSpec((_BQ, dk), lambda i: (i, 0)),
        pl.BlockSpec(w1k.shape, lambda i: (0, 0)),
        pl.BlockSpec(b1kt.shape, lambda i: (0, 0)),
        pl.BlockSpec(w2k.shape, lambda i: (0, 0)),
        pl.BlockSpec(b2kt.shape, lambda i: (0, 0)),
        pl.BlockSpec(sk.shape, lambda i: (0, 0)),
    ]
    args = (e4, w1k, b1kt, w2k, b2kt, sk)
    aliases = {}
    if acc is not None:
        specs = [pl.BlockSpec(memory_space=pl.ANY)] + specs
        args = (acc,) + args
        aliases = {0: 0}
    return pl.pallas_call(
        mlp_kernel,
        grid=(grid,),
        in_specs=specs,
        out_specs=pl.BlockSpec(
            (tags, _PACK * _BQ), lambda i: (0, i + blk0)
        ),
        out_shape=jax.ShapeDtypeStruct(out_shape, jnp.float32),
        input_output_aliases=aliases,
    )(*args)


def kernel(sentence, emb, fc_w, fc_b, out_w, out_b):
    n = sentence.shape[0]
    v, d = emb.shape
    h = fc_w.shape[0]
    tags = out_w.shape[0]
    hp = 64
    nc = n // _NCH

    idx = sentence.astype(jnp.int32)
    table128 = _tc_table_prep(emb.T, v, d)

    eye = jnp.eye(_PACK, dtype=jnp.float32)
    w1k = jnp.kron(eye, fc_w.T)
    b1kt = jnp.tile(fc_b, _PACK).reshape(_PACK * h, 1)
    w2p = jnp.pad(out_w.T, ((0, 0), (0, hp - tags)))
    w2k = jnp.kron(eye, w2p)
    b2kt = jnp.tile(jnp.pad(out_b, (0, hp - tags)), _PACK).reshape(_PACK * hp, 1)
    mask_ones = (jnp.arange(hp)[:, None] < tags).astype(jnp.float32)
    sk = jnp.kron(eye, jnp.broadcast_to(mask_ones, (hp, hp)))

    out_t = None
    for k in range(_NCH):
        embeds_k = _sc_gather_chunk(table128, idx, k, _NCH, d)
        e4_k = embeds_k.reshape(nc // _PACK, _PACK * d)
        out_t = _tc_mlp_chunk(
            e4_k, w1k, b1kt, w2k, b2kt, sk, tags, out_t, k * nc, (tags, n)
        )
    return out_t.T


# speedup vs baseline: 1.2296x; 1.2296x over previous
import functools

import jax
import jax.numpy as jnp
from jax import lax
from jax.experimental import pallas as pl
from jax.experimental.pallas import tpu as pltpu
from jax.experimental.pallas import tpu_sc as plsc

_NC = 2
_NS = 16
_NW = _NC * _NS

_SEG = 128
_GROUP_ROWS = 4 * _SEG

_PACK = 4
_BQ = 8192
_NCH = 5
_BT = 8192


def _tc_table_prep(embt, v, d):
    def prep_kernel(i_ref, o_ref):
        o_ref[:, :d] = jnp.transpose(i_ref[...])

    return pl.pallas_call(
        prep_kernel,
        grid=((v + _BT - 1) // _BT,),
        in_specs=[pl.BlockSpec((d, _BT), lambda i: (0, i))],
        out_specs=pl.BlockSpec((_BT, 128), lambda i: (i, 0)),
        out_shape=jax.ShapeDtypeStruct((v, 128), jnp.float32),
    )(embt)


def _sc_gather_chunk(table128, idx, chunk, n_chunks, d):
    n = idx.shape[0]
    q = n // _PACK
    n_groups = q // _SEG
    gpc = n_groups // n_chunks
    gpw = gpc // _NW
    g0 = chunk * gpc
    mesh = plsc.VectorSubcoreMesh(core_axis_name="c", subcore_axis_name="s")

    @functools.partial(
        pl.kernel,
        out_type=jax.ShapeDtypeStruct((n // n_chunks, d), jnp.float32),
        mesh=mesh,
        scratch_types=[
            pltpu.VMEM((_PACK * _SEG,), jnp.int32),
            pltpu.VMEM((_GROUP_ROWS,), jnp.int32),
            pltpu.VMEM((_GROUP_ROWS, 128), jnp.float32),
            pltpu.VMEM((_GROUP_ROWS, d), jnp.float32),
            pltpu.SemaphoreType.DMA,
            pltpu.SemaphoreType.DMA,
        ],
        compiler_params=pltpu.CompilerParams(
            use_tc_tiling_on_sc=False, needs_layout_passes=False
        ),
    )
    def gather_kernel(table_hbm, idx_hbm, out_hbm, idx_v, ilv_v, rows_v,
                      cmp_v, sem, isem):
        wid = lax.axis_index("s") * _NC + lax.axis_index("c")
        lane = lax.broadcasted_iota(jnp.int32, (16,), 0)
        half = d // 2

        def body(i, carry):
            gg = g0 + wid * gpw + i
            p0 = gg * _SEG
            blk = p0 // _BQ
            r0 = p0 % _BQ
            tok0 = blk * (_PACK * _BQ) + r0
            icopies = [
                pltpu.async_copy(
                    idx_hbm.at[pl.ds(tok0 + c * _BQ, _SEG)],
                    idx_v.at[pl.ds(c * _SEG, _SEG)],
                    isem,
                )
                for c in range(_PACK)
            ]
            for ic in icopies:
                ic.wait()
            for c in range(_PACK):
                for j in range(_SEG // 16):
                    vv = idx_v[pl.ds(c * _SEG + j * 16, 16)]
                    dst = (j * 16 + lane) * _PACK + c
                    plsc.store_scatter(ilv_v, [dst], vv)
            copies = [
                pltpu.async_copy(
                    table_hbm.at[ilv_v.at[pl.ds(b * 128, 128)]],
                    rows_v.at[pl.ds(b * 128, 128)],
                    sem,
                )
                for b in range(_GROUP_ROWS // 128)
            ]

            for b in range(_GROUP_ROWS // 128):
                copies[b].wait()

                def extract(j, carry2, b=b):
                    for u in range(8):
                        t = b * 128 + j * 8 + u
                        cmp_v[t, pl.ds(0, half)] = rows_v[t, pl.ds(0, half)]
                        cmp_v[t, pl.ds(half, half)] = (
                            rows_v[t, pl.ds(half, half)]
                        )
                    return carry2

                lax.fori_loop(0, 16, extract, 0)

            lp0 = (gg - g0) * _SEG
            pltpu.sync_copy(
                cmp_v, out_hbm.at[pl.ds(lp0 * _PACK, _GROUP_ROWS)]
            )
            return carry

        lax.fori_loop(0, gpw, body, 0)

    return gather_kernel(table128, idx)


def _tc_mlp_chunk(e4, w1k, b1kt, w2k, b2kt, sk, tags, acc, col0, out_shape):
    qc, dk = e4.shape
    hk = w1k.shape[1]
    hp = hk // _PACK
    grid = qc // _BQ
    blk0 = col0 // (_PACK * _BQ)

    def mlp_kernel(*refs):
        if acc is not None:
            refs = refs[1:]
        e_ref, w1_ref, b1_ref, w2_ref, b2_ref, s_ref, o_ref = refs
        e = e_ref[...]
        hidt = lax.dot_general(
            w1_ref[...], e, (((0,), (1,)), ((), ())),
            preferred_element_type=jnp.float32,
        )
        hidt = jnp.maximum(hidt + b1_ref[...], 0.0)
        t4t = lax.dot_general(
            w2_ref[...], hidt, (((0,), (0,)), ((), ())),
            preferred_element_type=jnp.float32,
        )
        t4t = t4t + b2_ref[...]
        m = jnp.max(t4t)
        ext = jnp.exp(t4t - m)
        sumst = lax.dot_general(
            s_ref[...], ext, (((0,), (0,)), ((), ())),
            preferred_element_type=jnp.float32,
        )
        rt = t4t - (m + jnp.log(sumst))
        for g in range(_PACK):
            o_ref[:, pl.ds(g * _BQ, _BQ)] = rt[g * hp:g * hp + tags, :]

    specs = [
        pl.BlockSpec((_BQ, dk), lambda i: (i, 0)),
        pl.BlockSpec(w1k.shape, lambda i: (0, 0)),
        pl.BlockSpec(b1kt.shape, lambda i: (0, 0)),
        pl.BlockSpec(w2k.shape, lambda i: (0, 0)),
        pl.BlockSpec(b2kt.shape, lambda i: (0, 0)),
        pl.BlockSpec(sk.shape, lambda i: (0, 0)),
    ]
    args = (e4, w1k, b1kt, w2k, b2kt, sk)
    aliases = {}
    if acc is not None:
        specs = [pl.BlockSpec(memory_space=pl.ANY)] + specs
        args = (acc,) + args
        aliases = {0: 0}
    return pl.pallas_call(
        mlp_kernel,
        grid=(grid,),
        in_specs=specs,
        out_specs=pl.BlockSpec(
            (tags, _PACK * _BQ), lambda i: (0, i + blk0)
        ),
        out_shape=jax.ShapeDtypeStruct(out_shape, jnp.float32),
        input_output_aliases=aliases,
    )(*args)


def kernel(sentence, emb, fc_w, fc_b, out_w, out_b):
    n = sentence.shape[0]
    v, d = emb.shape
    h = fc_w.shape[0]
    tags = out_w.shape[0]
    hp = 64
    nc = n // _NCH

    idx = sentence.astype(jnp.int32)
    table128 = _tc_table_prep(emb.T, v, d)

    eye = jnp.eye(_PACK, dtype=jnp.float32)
    w1k = jnp.kron(eye, fc_w.T)
    b1kt = jnp.tile(fc_b, _PACK).reshape(_PACK * h, 1)
    w2p = jnp.pad(out_w.T, ((0, 0), (0, hp - tags)))
    w2k = jnp.kron(eye, w2p)
    b2kt = jnp.tile(jnp.pad(out_b, (0, hp - tags)), _PACK).reshape(_PACK * hp, 1)
    mask_ones = (jnp.arange(hp)[:, None] < tags).astype(jnp.float32)
    sk = jnp.kron(eye, jnp.broadcast_to(mask_ones, (hp, hp)))

    out_t = None
    for k in range(_NCH):
        embeds_k = _sc_gather_chunk(table128, idx, k, _NCH, d)
        e4_k = embeds_k.reshape(nc // _PACK, _PACK * d)
        out_t = _tc_mlp_chunk(
            e4_k, w1k, b1kt, w2k, b2kt, sk, tags, out_t, k * nc, (tags, n)
        )
    return out_t.T


# speedup vs baseline: 1.2877x; 1.0473x over previous
import functools

import jax
import jax.numpy as jnp
from jax import lax
from jax.experimental import pallas as pl
from jax.experimental.pallas import tpu as pltpu
from jax.experimental.pallas import tpu_sc as plsc

_NC = 2
_NS = 16
_NW = _NC * _NS

_SEG = 128
_GROUP_ROWS = 4 * _SEG

_PACK = 4
_BQ = 8192
_NCH = 5
_BT = 16384


def _tc_table_prep(embt, v, d):
    def prep_kernel(i_ref, o_ref):
        o_ref[:, :d] = jnp.transpose(i_ref[...])

    return pl.pallas_call(
        prep_kernel,
        grid=((v + _BT - 1) // _BT,),
        in_specs=[pl.BlockSpec((d, _BT), lambda i: (0, i))],
        out_specs=pl.BlockSpec((_BT, 128), lambda i: (i, 0)),
        out_shape=jax.ShapeDtypeStruct((v, 128), jnp.float32),
    )(embt)


def _sc_gather_chunk(table128, idx, chunk, n_chunks, d):
    n = idx.shape[0]
    q = n // _PACK
    n_groups = q // _SEG
    gpc = n_groups // n_chunks
    gpw = gpc // _NW
    g0 = chunk * gpc
    mesh = plsc.VectorSubcoreMesh(core_axis_name="c", subcore_axis_name="s")

    @functools.partial(
        pl.kernel,
        out_type=jax.ShapeDtypeStruct((n // n_chunks, d), jnp.float32),
        mesh=mesh,
        scratch_types=[
            pltpu.VMEM((_PACK * _SEG,), jnp.int32),
            pltpu.VMEM((_GROUP_ROWS,), jnp.int32),
            pltpu.VMEM((_GROUP_ROWS, 128), jnp.float32),
            pltpu.VMEM((_GROUP_ROWS, d), jnp.float32),
            pltpu.SemaphoreType.DMA,
            pltpu.SemaphoreType.DMA,
        ],
        compiler_params=pltpu.CompilerParams(
            use_tc_tiling_on_sc=False, needs_layout_passes=False
        ),
    )
    def gather_kernel(table_hbm, idx_hbm, out_hbm, idx_v, ilv_v, rows_v,
                      cmp_v, sem, isem):
        wid = lax.axis_index("s") * _NC + lax.axis_index("c")
        lane = lax.broadcasted_iota(jnp.int32, (16,), 0)
        half = d // 2

        def body(i, carry):
            gg = g0 + wid * gpw + i
            p0 = gg * _SEG
            blk = p0 // _BQ
            r0 = p0 % _BQ
            tok0 = blk * (_PACK * _BQ) + r0
            icopies = [
                pltpu.async_copy(
                    idx_hbm.at[pl.ds(tok0 + c * _BQ, _SEG)],
                    idx_v.at[pl.ds(c * _SEG, _SEG)],
                    isem,
                )
                for c in range(_PACK)
            ]
            for ic in icopies:
                ic.wait()
            for c in range(_PACK):
                for j in range(_SEG // 16):
                    vv = idx_v[pl.ds(c * _SEG + j * 16, 16)]
                    dst = (j * 16 + lane) * _PACK + c
                    plsc.store_scatter(ilv_v, [dst], vv)
            copies = [
                pltpu.async_copy(
                    table_hbm.at[ilv_v.at[pl.ds(b * 128, 128)]],
                    rows_v.at[pl.ds(b * 128, 128)],
                    sem,
                )
                for b in range(_GROUP_ROWS // 128)
            ]

            for b in range(_GROUP_ROWS // 128):
                copies[b].wait()

                def extract(j, carry2, b=b):
                    for u in range(8):
                        t = b * 128 + j * 8 + u
                        cmp_v[t, pl.ds(0, half)] = rows_v[t, pl.ds(0, half)]
                        cmp_v[t, pl.ds(half, half)] = (
                            rows_v[t, pl.ds(half, half)]
                        )
                    return carry2

                lax.fori_loop(0, 16, extract, 0)

            lp0 = (gg - g0) * _SEG
            pltpu.sync_copy(
                cmp_v, out_hbm.at[pl.ds(lp0 * _PACK, _GROUP_ROWS)]
            )
            return carry

        lax.fori_loop(0, gpw, body, 0)

    return gather_kernel(table128, idx)


def _tc_mlp_chunk(e4, w1k, b1kt, w2k, b2kt, sk, tags, acc, col0, out_shape):
    qc, dk = e4.shape
    hk = w1k.shape[1]
    hp = hk // _PACK
    grid = qc // _BQ
    blk0 = col0 // (_PACK * _BQ)

    def mlp_kernel(*refs):
        if acc is not None:
            refs = refs[1:]
        e_ref, w1_ref, b1_ref, w2_ref, b2_ref, s_ref, o_ref = refs
        e = e_ref[...]
        hidt = lax.dot_general(
            w1_ref[...], e, (((0,), (1,)), ((), ())),
            preferred_element_type=jnp.float32,
        )
        hidt = jnp.maximum(hidt + b1_ref[...], 0.0)
        t4t = lax.dot_general(
            w2_ref[...], hidt, (((0,), (0,)), ((), ())),
            preferred_element_type=jnp.float32,
        )
        t4t = t4t + b2_ref[...]
        m = jnp.max(t4t)
        ext = jnp.exp(t4t - m)
        sumst = lax.dot_general(
            s_ref[...], ext, (((0,), (0,)), ((), ())),
            preferred_element_type=jnp.float32,
        )
        rt = t4t - (m + jnp.log(sumst))
        for g in range(_PACK):
            o_ref[:, pl.ds(g * _BQ, _BQ)] = rt[g * hp:g * hp + tags, :]

    specs = [
        pl.BlockSpec((_BQ, dk), lambda i: (i, 0)),
        pl.BlockSpec(w1k.shape, lambda i: (0, 0)),
        pl.BlockSpec(b1kt.shape, lambda i: (0, 0)),
        pl.BlockSpec(w2k.shape, lambda i: (0, 0)),
        pl.BlockSpec(b2kt.shape, lambda i: (0, 0)),
        pl.BlockSpec(sk.shape, lambda i: (0, 0)),
    ]
    args = (e4, w1k, b1kt, w2k, b2kt, sk)
    aliases = {}
    if acc is not None:
        specs = [pl.BlockSpec(memory_space=pl.ANY)] + specs
        args = (acc,) + args
        aliases = {0: 0}
    return pl.pallas_call(
        mlp_kernel,
        grid=(grid,),
        in_specs=specs,
        out_specs=pl.BlockSpec(
            (tags, _PACK * _BQ), lambda i: (0, i + blk0)
        ),
        out_shape=jax.ShapeDtypeStruct(out_shape, jnp.float32),
        input_output_aliases=aliases,
    )(*args)


def kernel(sentence, emb, fc_w, fc_b, out_w, out_b):
    n = sentence.shape[0]
    v, d = emb.shape
    h = fc_w.shape[0]
    tags = out_w.shape[0]
    hp = 64
    nc = n // _NCH

    idx = sentence.astype(jnp.int32)
    table128 = _tc_table_prep(emb.T, v, d)

    eye = jnp.eye(_PACK, dtype=jnp.float32)
    w1k = jnp.kron(eye, fc_w.T)
    b1kt = jnp.tile(fc_b, _PACK).reshape(_PACK * h, 1)
    w2p = jnp.pad(out_w.T, ((0, 0), (0, hp - tags)))
    w2k = jnp.kron(eye, w2p)
    b2kt = jnp.tile(jnp.pad(out_b, (0, hp - tags)), _PACK).reshape(_PACK * hp, 1)
    mask_ones = (jnp.arange(hp)[:, None] < tags).astype(jnp.float32)
    sk = jnp.kron(eye, jnp.broadcast_to(mask_ones, (hp, hp)))

    out_t = None
    for k in range(_NCH):
        embeds_k = _sc_gather_chunk(table128, idx, k, _NCH, d)
        e4_k = embeds_k.reshape(nc // _PACK, _PACK * d)
        out_t = _tc_mlp_chunk(
            e4_k, w1k, b1kt, w2k, b2kt, sk, tags, out_t, k * nc, (tags, n)
        )
    return out_t.T


# speedup vs baseline: 1.3271x; 1.0306x over previous
import functools

import jax
import jax.numpy as jnp
from jax import lax
from jax.experimental import pallas as pl
from jax.experimental.pallas import tpu as pltpu
from jax.experimental.pallas import tpu_sc as plsc

_NC = 2
_NS = 16
_NW = _NC * _NS

_SEG = 128
_GROUP_ROWS = 4 * _SEG

_PACK = 4
_BQ = 8192
_NCH = 5
_BT = 16384


def _tc_table_prep(embt, v, d):
    def prep_kernel(i_ref, o_ref):
        o_ref[:, :d] = jnp.transpose(i_ref[...])

    return pl.pallas_call(
        prep_kernel,
        grid=((v + _BT - 1) // _BT,),
        in_specs=[pl.BlockSpec((d, _BT), lambda i: (0, i))],
        out_specs=pl.BlockSpec((_BT, 128), lambda i: (i, 0)),
        out_shape=jax.ShapeDtypeStruct((v, 128), jnp.float32),
    )(embt)


def _sc_gather_chunk(table128, idx, chunk, n_chunks, d):
    n = idx.shape[0]
    q = n // _PACK
    n_groups = q // _SEG
    gpc = n_groups // n_chunks
    gpw = gpc // _NW
    g0 = chunk * gpc
    mesh = plsc.VectorSubcoreMesh(core_axis_name="c", subcore_axis_name="s")

    @functools.partial(
        pl.kernel,
        out_type=jax.ShapeDtypeStruct((n // n_chunks, d), jnp.float32),
        mesh=mesh,
        scratch_types=[
            pltpu.VMEM((_PACK * _SEG,), jnp.int32),
            pltpu.VMEM((_GROUP_ROWS,), jnp.int32),
            pltpu.VMEM((_GROUP_ROWS, 128), jnp.float32),
            pltpu.VMEM((_GROUP_ROWS, d), jnp.float32),
            pltpu.SemaphoreType.DMA,
            pltpu.SemaphoreType.DMA,
            pltpu.SemaphoreType.DMA,
        ],
        compiler_params=pltpu.CompilerParams(
            use_tc_tiling_on_sc=False, needs_layout_passes=False
        ),
    )
    def gather_kernel(table_hbm, idx_hbm, out_hbm, idx_v, ilv_v, rows_v,
                      cmp_v, sem, isem, osem):
        wid = lax.axis_index("s") * _NC + lax.axis_index("c")
        lane = lax.broadcasted_iota(jnp.int32, (16,), 0)
        half = d // 2

        def body(i, carry):
            gg = g0 + wid * gpw + i
            p0 = gg * _SEG
            blk = p0 // _BQ
            r0 = p0 % _BQ
            tok0 = blk * (_PACK * _BQ) + r0
            icopies = [
                pltpu.async_copy(
                    idx_hbm.at[pl.ds(tok0 + c * _BQ, _SEG)],
                    idx_v.at[pl.ds(c * _SEG, _SEG)],
                    isem,
                )
                for c in range(_PACK)
            ]
            for ic in icopies:
                ic.wait()
            for c in range(_PACK):
                for j in range(_SEG // 16):
                    vv = idx_v[pl.ds(c * _SEG + j * 16, 16)]
                    dst = (j * 16 + lane) * _PACK + c
                    plsc.store_scatter(ilv_v, [dst], vv)
            copies = [
                pltpu.async_copy(
                    table_hbm.at[ilv_v.at[pl.ds(b * 128, 128)]],
                    rows_v.at[pl.ds(b * 128, 128)],
                    sem,
                )
                for b in range(_GROUP_ROWS // 128)
            ]

            @pl.when(i > 0)
            def _wait_prev_out():
                pltpu.make_async_copy(
                    cmp_v, out_hbm.at[pl.ds(0, _GROUP_ROWS)], osem
                ).wait()

            for b in range(_GROUP_ROWS // 128):
                copies[b].wait()

                def extract(j, carry2, b=b):
                    for u in range(8):
                        t = b * 128 + j * 8 + u
                        cmp_v[t, pl.ds(0, half)] = rows_v[t, pl.ds(0, half)]
                        cmp_v[t, pl.ds(half, half)] = (
                            rows_v[t, pl.ds(half, half)]
                        )
                    return carry2

                lax.fori_loop(0, 16, extract, 0)

            lp0 = (gg - g0) * _SEG
            pltpu.async_copy(
                cmp_v, out_hbm.at[pl.ds(lp0 * _PACK, _GROUP_ROWS)], osem
            )
            return carry

        lax.fori_loop(0, gpw, body, 0)
        pltpu.make_async_copy(
            cmp_v, out_hbm.at[pl.ds(0, _GROUP_ROWS)], osem
        ).wait()

    return gather_kernel(table128, idx)


def _tc_mlp_chunk(e4, w1k, b1kt, w2k, b2kt, sk, tags, acc, col0, out_shape):
    qc, dk = e4.shape
    hk = w1k.shape[1]
    hp = hk // _PACK
    grid = qc // _BQ
    blk0 = col0 // (_PACK * _BQ)

    def mlp_kernel(*refs):
        if acc is not None:
            refs = refs[1:]
        e_ref, w1_ref, b1_ref, w2_ref, b2_ref, s_ref, o_ref = refs
        e = e_ref[...]
        hidt = lax.dot_general(
            w1_ref[...], e, (((0,), (1,)), ((), ())),
            preferred_element_type=jnp.float32,
        )
        hidt = jnp.maximum(hidt + b1_ref[...], 0.0)
        t4t = lax.dot_general(
            w2_ref[...], hidt, (((0,), (0,)), ((), ())),
            preferred_element_type=jnp.float32,
        )
        t4t = t4t + b2_ref[...]
        m = jnp.max(t4t)
        ext = jnp.exp(t4t - m)
        sumst = lax.dot_general(
            s_ref[...], ext, (((0,), (0,)), ((), ())),
            preferred_element_type=jnp.float32,
        )
        rt = t4t - (m + jnp.log(sumst))
        for g in range(_PACK):
            o_ref[:, pl.ds(g * _BQ, _BQ)] = rt[g * hp:g * hp + tags, :]

    specs = [
        pl.BlockSpec((_BQ, dk), lambda i: (i, 0)),
        pl.BlockSpec(w1k.shape, lambda i: (0, 0)),
        pl.BlockSpec(b1kt.shape, lambda i: (0, 0)),
        pl.BlockSpec(w2k.shape, lambda i: (0, 0)),
        pl.BlockSpec(b2kt.shape, lambda i: (0, 0)),
        pl.BlockSpec(sk.shape, lambda i: (0, 0)),
    ]
    args = (e4, w1k, b1kt, w2k, b2kt, sk)
    aliases = {}
    if acc is not None:
        specs = [pl.BlockSpec(memory_space=pl.ANY)] + specs
        args = (acc,) + args
        aliases = {0: 0}
    return pl.pallas_call(
        mlp_kernel,
        grid=(grid,),
        in_specs=specs,
        out_specs=pl.BlockSpec(
            (tags, _PACK * _BQ), lambda i: (0, i + blk0)
        ),
        out_shape=jax.ShapeDtypeStruct(out_shape, jnp.float32),
        input_output_aliases=aliases,
    )(*args)


def kernel(sentence, emb, fc_w, fc_b, out_w, out_b):
    n = sentence.shape[0]
    v, d = emb.shape
    h = fc_w.shape[0]
    tags = out_w.shape[0]
    hp = 64
    nc = n // _NCH

    idx = sentence.astype(jnp.int32)
    table128 = _tc_table_prep(emb.T, v, d)

    eye = jnp.eye(_PACK, dtype=jnp.float32)
    w1k = jnp.kron(eye, fc_w.T)
    b1kt = jnp.tile(fc_b, _PACK).reshape(_PACK * h, 1)
    w2p = jnp.pad(out_w.T, ((0, 0), (0, hp - tags)))
    w2k = jnp.kron(eye, w2p)
    b2kt = jnp.tile(jnp.pad(out_b, (0, hp - tags)), _PACK).reshape(_PACK * hp, 1)
    mask_ones = (jnp.arange(hp)[:, None] < tags).astype(jnp.float32)
    sk = jnp.kron(eye, jnp.broadcast_to(mask_ones, (hp, hp)))

    out_t = None
    for k in range(_NCH):
        embeds_k = _sc_gather_chunk(table128, idx, k, _NCH, d)
        e4_k = embeds_k.reshape(nc // _PACK, _PACK * d)
        out_t = _tc_mlp_chunk(
            e4_k, w1k, b1kt, w2k, b2kt, sk, tags, out_t, k * nc, (tags, n)
        )
    return out_t.T
